# baseline (speedup 1.0000x reference)
import functools

import jax
import jax.numpy as jnp
from jax import lax
from jax.experimental import pallas as pl
from jax.experimental.pallas import tpu as pltpu
from jax.experimental.pallas import tpu_sc as plsc

N = 10000
E = 320000
H = 8
OUT = 16
NC = 2
NS = 16
LANES = 16
N_PAD = 10240
E_PER_W = E // (NC * NS)
CHUNK = 128
CH = -(-E_PER_W // CHUNK)
E_PAD_W = CH * CHUNK


def _sc_degree_kernel():
    mesh = plsc.VectorSubcoreMesh(core_axis_name="c", subcore_axis_name="s")

    @functools.partial(
        pl.kernel,
        mesh=mesh,
        out_type=jax.ShapeDtypeStruct((NC, N_PAD), jnp.float32),
        scratch_types=[
            pltpu.VMEM((CH, CHUNK), jnp.int32),
            pltpu.VMEM((CHUNK,), jnp.float32),
            pltpu.VMEM_SHARED((N_PAD,), jnp.float32),
        ],
    )
    def sc_deg(dst_hbm, zeros_hbm, out_hbm, idx_v, ones_v, deg_sh):
        c = lax.axis_index("c")
        s = lax.axis_index("s")

        for i in range(CHUNK // LANES):
            ones_v[pl.ds(i * LANES, LANES)] = jnp.full(
                (LANES,), 1.0, jnp.float32)

        @pl.when(s == 0)
        def _():
            pltpu.sync_copy(zeros_hbm, deg_sh)

        plsc.subcore_barrier()

        pltpu.sync_copy(dst_hbm.at[c, s], idx_v)

        def body(j, carry):
            pltpu.sync_copy(ones_v, deg_sh.at[idx_v.at[j]], add=True)
            return carry

        lax.fori_loop(0, CH, body, 0)

        plsc.subcore_barrier()

        @pl.when(s == 0)
        def _():
            pltpu.sync_copy(deg_sh, out_hbm.at[c])

    return sc_deg


def _tc_body(ft_ref, deg_ref, bias_ref, out_ref):
    d = deg_ref[...]
    mask = (d[:, 0:1] + d[:, 1:2]) > 0.0
    x = ft_ref[...]
    row = lax.broadcasted_iota(jnp.int32, (H * OUT, OUT), 0)
    col = lax.broadcasted_iota(jnp.int32, (H * OUT, OUT), 1)
    s = jnp.where(row % OUT == col, 1.0 / H, 0.0)
    acc = jnp.dot(x, s, preferred_element_type=jnp.float32,
                  precision=lax.Precision.HIGHEST)
    bias_mean = jnp.mean(bias_ref[...], axis=0, keepdims=True)
    out_ref[...] = jnp.where(mask, acc, 0.0) + bias_mean


def kernel(ft, e_ft, edge_index, W, bias):
    del e_ft, W
    n, h, out = ft.shape

    dst = edge_index[1]
    dst_pad = jnp.concatenate(
        [dst, jnp.full((NC * NS * E_PAD_W - E,), N, jnp.int32)]
    ).reshape(NC, NS, CH, CHUNK)
    zeros = jnp.zeros((N_PAD,), jnp.float32)

    deg2 = _sc_degree_kernel()(dst_pad, zeros)
    deg_t = jnp.swapaxes(deg2, 0, 1)[:n]

    bias2 = bias.reshape(h, out)

    return pl.pallas_call(
        _tc_body,
        out_shape=jax.ShapeDtypeStruct((n, out), jnp.float32),
    )(ft.reshape(n, h * out), deg_t, bias2)


# speedup vs baseline: 1.0538x; 1.0538x over previous
import functools

import jax
import jax.numpy as jnp
from jax import lax
from jax.experimental import pallas as pl
from jax.experimental.pallas import tpu as pltpu
from jax.experimental.pallas import tpu_sc as plsc

N = 10000
E = 320000
H = 8
OUT = 16
NC = 2
NS = 16
LANES = 16
N_PAD = 10240
E_PER_W = E // (NC * NS)
CHUNK = 128
CH = -(-E_PER_W // CHUNK)
E_PAD_W = CH * CHUNK


def _sc_degree_kernel():
    mesh = plsc.VectorSubcoreMesh(core_axis_name="c", subcore_axis_name="s")

    @functools.partial(
        pl.kernel,
        mesh=mesh,
        out_type=jax.ShapeDtypeStruct((NC, N_PAD), jnp.float32),
        scratch_types=[
            pltpu.VMEM((CH, CHUNK), jnp.int32),
            pltpu.VMEM((CHUNK,), jnp.float32),
            pltpu.VMEM_SHARED((N_PAD,), jnp.float32),
            pltpu.SemaphoreType.DMA,
            pltpu.SemaphoreType.DMA,
        ],
    )
    def sc_deg(dst_hbm, zeros_hbm, out_hbm, idx_v, ones_v, deg_sh,
               sem_idx, sem_sc):
        c = lax.axis_index("c")
        s = lax.axis_index("s")
        sl = N_PAD // NS

        h_idx = pltpu.async_copy(dst_hbm.at[c, s], idx_v, sem_idx)

        for i in range(CHUNK // LANES):
            ones_v[pl.ds(i * LANES, LANES)] = jnp.full(
                (LANES,), 1.0, jnp.float32)

        pltpu.sync_copy(zeros_hbm.at[pl.ds(s * sl, sl)],
                        deg_sh.at[pl.ds(s * sl, sl)])
        plsc.subcore_barrier()
        h_idx.wait()

        def fire(j, carry):
            pltpu.async_copy(ones_v, deg_sh.at[idx_v.at[j]], sem_sc,
                             add=True)
            return carry

        lax.fori_loop(0, CH, fire, 0)

        def drain(j, carry):
            pltpu.make_async_copy(ones_v, deg_sh.at[idx_v.at[j]],
                                  sem_sc).wait()
            return carry

        lax.fori_loop(0, CH, drain, 0)
        plsc.subcore_barrier()

        pltpu.sync_copy(deg_sh.at[pl.ds(s * sl, sl)],
                        out_hbm.at[c, pl.ds(s * sl, sl)])

    return sc_deg


def _tc_body(ft_ref, deg_ref, bias_ref, out_ref):
    d = deg_ref[...]
    mask = (d[:, 0:1] + d[:, 1:2]) > 0.0
    x = ft_ref[...]
    row = lax.broadcasted_iota(jnp.int32, (H * OUT, OUT), 0)
    col = lax.broadcasted_iota(jnp.int32, (H * OUT, OUT), 1)
    s = jnp.where(row % OUT == col, 1.0 / H, 0.0)
    acc = jnp.dot(x, s, preferred_element_type=jnp.float32,
                  precision=lax.Precision.HIGHEST)
    bias_mean = jnp.mean(bias_ref[...], axis=0, keepdims=True)
    out_ref[...] = jnp.where(mask, acc, 0.0) + bias_mean


def kernel(ft, e_ft, edge_index, W, bias):
    del e_ft, W
    n, h, out = ft.shape

    dst = edge_index[1]
    dst_pad = jnp.concatenate(
        [dst, jnp.full((NC * NS * E_PAD_W - E,), N, jnp.int32)]
    ).reshape(NC, NS, CH, CHUNK)
    zeros = jnp.zeros((N_PAD,), jnp.float32)

    deg2 = _sc_degree_kernel()(dst_pad, zeros)
    deg_t = jnp.swapaxes(deg2, 0, 1)[:n]

    bias2 = bias.reshape(h, out)

    return pl.pallas_call(
        _tc_body,
        out_shape=jax.ShapeDtypeStruct((n, out), jnp.float32),
    )(ft.reshape(n, h * out), deg_t, bias2)


# speedup vs baseline: 1.8076x; 1.7153x over previous
import functools

import jax
import jax.numpy as jnp
from jax import lax
from jax.experimental import pallas as pl
from jax.experimental.pallas import tpu as pltpu
from jax.experimental.pallas import tpu_sc as plsc

N = 10000
E = 320000
H = 8
OUT = 16
NC = 2
NS = 16
LANES = 16
N_PAD = 10240
E_PER_W = E // (NC * NS)
CHUNK = 128
FULL = E_PER_W // CHUNK
TAIL = E_PER_W - FULL * CHUNK
CH = FULL + 1


def _sc_degree_kernel():
    mesh = plsc.VectorSubcoreMesh(core_axis_name="c", subcore_axis_name="s")

    @functools.partial(
        pl.kernel,
        mesh=mesh,
        out_type=jax.ShapeDtypeStruct((NC, N_PAD), jnp.float32),
        scratch_types=[
            pltpu.VMEM((CH, CHUNK), jnp.int32),
            pltpu.VMEM((CHUNK,), jnp.float32),
            pltpu.VMEM((N_PAD // NS,), jnp.float32),
            pltpu.VMEM_SHARED((N_PAD,), jnp.float32),
            pltpu.SemaphoreType.DMA,
            pltpu.SemaphoreType.DMA,
        ],
    )
    def sc_deg(ei_hbm, out_hbm, idx_v, ones_v, zero_v, deg_sh,
               sem_idx, sem_sc):
        c = lax.axis_index("c")
        s = lax.axis_index("s")
        w = c * NS + s
        sl = N_PAD // NS

        h_idx = pltpu.async_copy(ei_hbm.at[1, w], idx_v, sem_idx)

        for i in range(CHUNK // LANES):
            ones_v[pl.ds(i * LANES, LANES)] = jnp.full(
                (LANES,), 1.0, jnp.float32)
        for i in range(sl // LANES):
            zero_v[pl.ds(i * LANES, LANES)] = jnp.zeros((LANES,), jnp.float32)
        pltpu.sync_copy(zero_v, deg_sh.at[pl.ds(s * sl, sl)])

        h_idx.wait()
        plsc.subcore_barrier()

        def fire(j, carry):
            pltpu.async_copy(ones_v, deg_sh.at[idx_v.at[j]], sem_sc,
                             add=True)
            return carry

        lax.fori_loop(0, CH, fire, 0)

        def drain(j, carry):
            pltpu.make_async_copy(ones_v, deg_sh.at[idx_v.at[j]],
                                  sem_sc).wait()
            return carry

        lax.fori_loop(0, CH, drain, 0)
        plsc.subcore_barrier()

        pltpu.sync_copy(deg_sh.at[pl.ds(s * sl, sl)],
                        out_hbm.at[c, pl.ds(s * sl, sl)])

    return sc_deg


def _tc_body(xt_ref, deg_ref, biast_ref, out_ref):
    d = deg_ref[...]
    mask = (d[0:1, :N] + d[1:2, :N]) > 0.0
    row = lax.broadcasted_iota(jnp.int32, (OUT, H * OUT), 0)
    col = lax.broadcasted_iota(jnp.int32, (OUT, H * OUT), 1)
    s = jnp.where(col % OUT == row, 1.0 / H, 0.0)
    acc = jnp.dot(s, xt_ref[...], preferred_element_type=jnp.float32,
                  precision=lax.Precision.HIGHEST)
    bias_mean = jnp.mean(biast_ref[...], axis=1, keepdims=True)
    out_ref[...] = jnp.where(mask, acc, 0.0) + bias_mean


def kernel(ft, e_ft, edge_index, W, bias):
    del e_ft, W
    n, h, out = ft.shape

    ei4 = jnp.pad(edge_index, ((0, 0), (0, NC * NS * CH * CHUNK - E)),
                  constant_values=N).reshape(2, NC * NS, CH, CHUNK)
    deg2 = _sc_degree_kernel()(ei4)

    xt = jnp.transpose(ft, (1, 2, 0)).reshape(h * out, n)
    biast = jnp.swapaxes(bias.reshape(h, out), 0, 1)

    out_t = pl.pallas_call(
        _tc_body,
        out_shape=jax.ShapeDtypeStruct((out, n), jnp.float32),
    )(xt, deg2, biast)
    return jnp.swapaxes(out_t, 0, 1)


# speedup vs baseline: 2.4490x; 1.3548x over previous
import functools

import jax
import jax.numpy as jnp
from jax import lax
from jax.experimental import pallas as pl
from jax.experimental.pallas import tpu as pltpu
from jax.experimental.pallas import tpu_sc as plsc

N = 10000
E = 320000
H = 8
OUT = 16
NC = 2
NS = 16
NW = NC * NS
LANES = 16
N_PAD = 10240
CHUNK = 128
NCHUNKS = E // CHUNK
BASE_CH = NCHUNKS // NW
EXTRA = NCHUNKS - BASE_CH * NW
MAX_CH = BASE_CH + 1


def _sc_degree_kernel():
    mesh = plsc.VectorSubcoreMesh(core_axis_name="c", subcore_axis_name="s")

    @functools.partial(
        pl.kernel,
        mesh=mesh,
        out_type=jax.ShapeDtypeStruct((NC, N_PAD), jnp.float32),
        scratch_types=[
            pltpu.VMEM((MAX_CH, 2, CHUNK), jnp.int32),
            pltpu.VMEM((CHUNK,), jnp.float32),
            pltpu.VMEM((N_PAD // NS,), jnp.float32),
            pltpu.VMEM_SHARED((N_PAD,), jnp.float32),
            pltpu.SemaphoreType.DMA,
            pltpu.SemaphoreType.DMA,
        ],
    )
    def sc_deg(ei_hbm, out_hbm, idx_v, ones_v, zero_v, deg_sh,
               sem_idx, sem_sc):
        c = lax.axis_index("c")
        s = lax.axis_index("s")
        w = c * NS + s
        nch = jnp.where(w < EXTRA, BASE_CH + 1, BASE_CH)
        base = w * BASE_CH + jnp.minimum(w, EXTRA)
        sl = N_PAD // NS

        def fire_load(j, carry):
            pltpu.async_copy(
                ei_hbm.at[:, pl.ds((base + j) * CHUNK, CHUNK)],
                idx_v.at[j], sem_idx)
            return carry

        lax.fori_loop(0, nch, fire_load, 0)

        for i in range(CHUNK // LANES):
            ones_v[pl.ds(i * LANES, LANES)] = jnp.full(
                (LANES,), 1.0, jnp.float32)
        for i in range(sl // LANES):
            zero_v[pl.ds(i * LANES, LANES)] = jnp.zeros((LANES,), jnp.float32)
        pltpu.sync_copy(zero_v, deg_sh.at[pl.ds(s * sl, sl)])

        def drain_load(j, carry):
            pltpu.make_async_copy(
                ei_hbm.at[:, pl.ds((base + j) * CHUNK, CHUNK)],
                idx_v.at[j], sem_idx).wait()
            return carry

        lax.fori_loop(0, nch, drain_load, 0)
        plsc.subcore_barrier()

        def fire(j, carry):
            pltpu.async_copy(ones_v, deg_sh.at[idx_v.at[j, 1]], sem_sc,
                             add=True)
            return carry

        lax.fori_loop(0, nch, fire, 0)

        def drain(j, carry):
            pltpu.make_async_copy(ones_v, deg_sh.at[idx_v.at[j, 1]],
                                  sem_sc).wait()
            return carry

        lax.fori_loop(0, nch, drain, 0)
        plsc.subcore_barrier()

        pltpu.sync_copy(deg_sh.at[pl.ds(s * sl, sl)],
                        out_hbm.at[c, pl.ds(s * sl, sl)])

    return sc_deg


def _mean_body(xt_ref, out_ref):
    row = lax.broadcasted_iota(jnp.int32, (OUT, H * OUT), 0)
    col = lax.broadcasted_iota(jnp.int32, (OUT, H * OUT), 1)
    s = jnp.where(col % OUT == row, 1.0 / H, 0.0)
    out_ref[...] = jnp.dot(s, xt_ref[...],
                           preferred_element_type=jnp.float32,
                           precision=lax.Precision.HIGHEST)


def _apply_body(acc_ref, deg_ref, biast_ref, out_ref):
    d = deg_ref[...]
    mask = (d[0:1, :N] + d[1:2, :N]) > 0.0
    bias_mean = jnp.mean(biast_ref[...], axis=1, keepdims=True)
    out_ref[...] = jnp.where(mask, acc_ref[...], 0.0) + bias_mean


def kernel(ft, e_ft, edge_index, W, bias):
    del e_ft, W
    n, h, out = ft.shape

    deg2 = _sc_degree_kernel()(edge_index)

    xt = jnp.transpose(ft, (1, 2, 0)).reshape(h * out, n)
    biast = jnp.swapaxes(bias.reshape(h, out), 0, 1)

    acc_t = pl.pallas_call(
        _mean_body,
        out_shape=jax.ShapeDtypeStruct((out, n), jnp.float32),
    )(xt)
    out_t = pl.pallas_call(
        _apply_body,
        out_shape=jax.ShapeDtypeStruct((out, n), jnp.float32),
    )(acc_t, deg2, biast)
    return jnp.swapaxes(out_t, 0, 1)


# speedup vs baseline: 2.5333x; 1.0345x over previous
import functools

import jax
import jax.numpy as jnp
from jax import lax
from jax.experimental import pallas as pl
from jax.experimental.pallas import tpu as pltpu
from jax.experimental.pallas import tpu_sc as plsc

N = 10000
E = 320000
H = 8
OUT = 16
NC = 2
NS = 16
NW = NC * NS
LANES = 16
N_PAD = 10240
CHUNK = 128
NCHUNKS = E // CHUNK
BASE_CH = NCHUNKS // NW
EXTRA = NCHUNKS - BASE_CH * NW
MAX_CH = BASE_CH + 1


def _sc_degree_kernel():
    mesh = plsc.VectorSubcoreMesh(core_axis_name="c", subcore_axis_name="s")

    @functools.partial(
        pl.kernel,
        mesh=mesh,
        out_type=jax.ShapeDtypeStruct((NC, N_PAD), jnp.float32),
        scratch_types=[
            pltpu.VMEM((MAX_CH, 2, CHUNK), jnp.int32),
            pltpu.VMEM((CHUNK,), jnp.float32),
            pltpu.VMEM((N_PAD // NS,), jnp.float32),
            pltpu.VMEM_SHARED((N_PAD,), jnp.float32),
            pltpu.SemaphoreType.DMA,
            pltpu.SemaphoreType.DMA,
        ],
    )
    def sc_deg(ei_hbm, out_hbm, idx_v, ones_v, zero_v, deg_sh,
               sem_idx, sem_sc):
        c = lax.axis_index("c")
        s = lax.axis_index("s")
        w = c * NS + s
        nch = jnp.where(w < EXTRA, BASE_CH + 1, BASE_CH)
        base = w * BASE_CH + jnp.minimum(w, EXTRA)
        sl = N_PAD // NS

        def fire_load(j, carry):
            pltpu.async_copy(
                ei_hbm.at[:, pl.ds((base + j) * CHUNK, CHUNK)],
                idx_v.at[j], sem_idx)
            return carry

        lax.fori_loop(0, nch, fire_load, 0)

        for i in range(CHUNK // LANES):
            ones_v[pl.ds(i * LANES, LANES)] = jnp.full(
                (LANES,), 1.0, jnp.float32)
        for i in range(sl // LANES):
            zero_v[pl.ds(i * LANES, LANES)] = jnp.zeros((LANES,), jnp.float32)
        pltpu.sync_copy(zero_v, deg_sh.at[pl.ds(s * sl, sl)])

        plsc.subcore_barrier()

        def land_and_fire(j, carry):
            pltpu.make_async_copy(
                ei_hbm.at[:, pl.ds((base + j) * CHUNK, CHUNK)],
                idx_v.at[j], sem_idx).wait()
            pltpu.async_copy(ones_v, deg_sh.at[idx_v.at[j, 1]], sem_sc,
                             add=True)
            return carry

        lax.fori_loop(0, nch, land_and_fire, 0)

        def drain(j, carry):
            pltpu.make_async_copy(ones_v, deg_sh.at[idx_v.at[j, 1]],
                                  sem_sc).wait()
            return carry

        lax.fori_loop(0, nch, drain, 0)
        plsc.subcore_barrier()

        pltpu.sync_copy(deg_sh.at[pl.ds(s * sl, sl)],
                        out_hbm.at[c, pl.ds(s * sl, sl)])

    return sc_deg


def _mean_body(xt_ref, out_ref):
    row = lax.broadcasted_iota(jnp.int32, (OUT, H * OUT), 0)
    col = lax.broadcasted_iota(jnp.int32, (OUT, H * OUT), 1)
    s = jnp.where(col % OUT == row, 1.0 / H, 0.0)
    out_ref[...] = jnp.dot(s, xt_ref[...],
                           preferred_element_type=jnp.float32,
                           precision=lax.Precision.HIGHEST)


def _apply_body(acc_ref, deg_ref, biast_ref, out_ref):
    d = deg_ref[...]
    mask = (d[0:1, :N] + d[1:2, :N]) > 0.0
    bias_mean = jnp.mean(biast_ref[...], axis=1, keepdims=True)
    out_ref[...] = jnp.where(mask, acc_ref[...], 0.0) + bias_mean


def kernel(ft, e_ft, edge_index, W, bias):
    del e_ft, W
    n, h, out = ft.shape

    deg2 = _sc_degree_kernel()(edge_index)

    xt = jnp.transpose(ft, (1, 2, 0)).reshape(h * out, n)
    biast = jnp.swapaxes(bias.reshape(h, out), 0, 1)

    acc_t = pl.pallas_call(
        _mean_body,
        out_shape=jax.ShapeDtypeStruct((out, n), jnp.float32),
    )(xt)
    out_t = pl.pallas_call(
        _apply_body,
        out_shape=jax.ShapeDtypeStruct((out, n), jnp.float32),
    )(acc_t, deg2, biast)
    return jnp.swapaxes(out_t, 0, 1)
